# Initial kernel scaffold; baseline (speedup 1.0000x reference)
import jax
import jax.numpy as jnp
from jax.experimental import pallas as pl


def kernel(x, edge_index, W1, b1, W2, b2):
    raise NotImplementedError("write your pallas kernel here")


# speedup vs baseline: 1.0437x; 1.0437x over previous
import jax
import jax.numpy as jnp
from jax.experimental import pallas as pl
from jax.experimental.pallas import tpu as pltpu

_UTSD = 8
_MAXD = 32
_K = 4
_M = _MAXD + 1
_BIG = 1e9
_B = 8


def _uts_body(htab_ref, x_ref, w1_ref, b1_ref, w2_ref, b2_ref, out_ref, h_ref):
    for b in range(_B):
        for j in range(_M):
            idx = htab_ref[0, b, j]
            h_ref[pl.ds(b * _M + j, 1), :] = x_ref[pl.ds(idx, 1), :]

    ii = jax.lax.broadcasted_iota(jnp.int32, (_M, _M), 0)
    jj = jax.lax.broadcasted_iota(jnp.int32, (_M, _M), 1)
    iota_col = jax.lax.broadcasted_iota(jnp.int32, (_M, 1), 0)

    uts_rows = []
    for b in range(_B):
        h = h_ref[pl.ds(b * _M, _M), :]
        gram = jax.lax.dot_general(
            h, h, (((1,), (1,)), ((), ())), preferred_element_type=jnp.float32
        )
        eye = (ii == jj).astype(jnp.float32)
        diag_col = jnp.sum(gram * eye, axis=1, keepdims=True)
        diag_row = jnp.sum(gram * eye, axis=0, keepdims=True)
        d2 = diag_col + diag_row - 2.0 * gram
        dist = jnp.sqrt(jnp.maximum(d2, 0.0) + 1e-12)

        c = htab_ref[0, b, _M]
        mask_row = jj < c
        mask_col_b = iota_col < c
        pair_ok = (ii < c) & mask_row & (ii != jj)
        dm = jnp.where(pair_ok, dist, _BIG)

        kds = []
        for k in range(_K):
            m = jnp.min(dm, axis=1, keepdims=True)
            kds.append(m)
            if k < _K - 1:
                sel = jnp.where(dm <= m, jj, _M)
                am = jnp.min(sel, axis=1, keepdims=True)
                dm = jnp.where(jj == am, _BIG, dm)
        kd = jnp.concatenate(kds, axis=1)

        w = (kd < _BIG * 0.5) & mask_col_b
        wf = w.astype(jnp.float32)
        wsum = jnp.maximum(jnp.sum(wf), 1.0)
        kdw = wf * kd
        s_mean = jnp.sum(kdw) / wsum
        s_var = jnp.sum(wf * (kd - s_mean) ** 2) / wsum
        s_std = jnp.sqrt(s_var + 1e-12)
        s_min = jnp.min(jnp.where(w, kd, _BIG))
        s_max = jnp.max(jnp.where(w, kd, -_BIG))

        pk = jnp.maximum(jnp.sum(wf, axis=1, keepdims=True), 1.0)
        pm = jnp.sum(kdw, axis=1, keepdims=True) / pk
        mfc = mask_col_b.astype(jnp.float32)
        cf = jnp.maximum(c.astype(jnp.float32), 1.0)
        p_mean = jnp.sum(mfc * pm) / cf
        p_std = jnp.sqrt(jnp.sum(mfc * (pm - p_mean) ** 2) / cf + 1e-12)

        nd = kd[:, 0:1]
        ndokf = mfc * (nd < _BIG * 0.5).astype(jnp.float32)
        n_mean = jnp.sum(ndokf * nd) / cf
        n_std = jnp.sqrt(jnp.sum(ndokf * (nd - n_mean) ** 2) / cf + 1e-12)

        okf = jnp.where(c >= 3, 1.0, 0.0)
        vals = [s_mean, s_std, s_min, s_max, p_mean, p_std, n_mean, n_std]
        row = jnp.concatenate(
            [jnp.broadcast_to(v * okf, (1, 1)) for v in vals], axis=1
        )
        uts_rows.append(row)

    uts = jnp.concatenate(uts_rows, axis=0)
    h1 = jax.lax.dot_general(
        uts, w1_ref[...], (((1,), (0,)), ((), ())), preferred_element_type=jnp.float32
    )
    h1 = jnp.maximum(h1 + b1_ref[...], 0.0)
    sc = jax.lax.dot_general(
        h1, w2_ref[...], (((1,), (0,)), ((), ())), preferred_element_type=jnp.float32
    )
    out_ref[0] = sc + b2_ref[...]


def kernel(x, edge_index, W1, b1, W2, b2):
    N, D = x.shape
    E = edge_index.shape[1]
    ei = edge_index.astype(jnp.int32)
    s_all = jnp.concatenate([ei[0], ei[1]])
    d_all = jnp.concatenate([ei[1], ei[0]])
    ss, dd = jax.lax.sort((s_all, d_all), num_keys=1, is_stable=True)
    counts = jnp.bincount(s_all, length=N).astype(jnp.int32)
    starts = (jnp.cumsum(counts) - counts).astype(jnp.int32)
    rank = jnp.arange(2 * E, dtype=jnp.int32) - starts[ss]
    valid = rank < _MAXD
    row = jnp.where(valid, ss, N)
    col = jnp.where(valid, jnp.minimum(rank, _MAXD - 1), 0)
    nbr = (
        jnp.zeros((N + 1, _MAXD), dtype=jnp.int32)
        .at[row, col]
        .set(jnp.where(valid, dd, 0))[:N]
    )
    cnt = 1 + jnp.minimum(counts, _MAXD)
    hood = jnp.concatenate(
        [jnp.arange(N, dtype=jnp.int32)[:, None], nbr, cnt[:, None]], axis=1
    )

    pad = (-N) % _B
    if pad:
        filler = jnp.zeros((pad, _M + 1), dtype=jnp.int32).at[:, _M].set(1)
        hood = jnp.concatenate([hood, filler], axis=0)
    nblk = hood.shape[0] // _B
    htab = hood.reshape(nblk, _B, _M + 1)

    scores = pl.pallas_call(
        _uts_body,
        grid=(nblk,),
        in_specs=[
            pl.BlockSpec(
                (1, _B, _M + 1), lambda i: (i, 0, 0), memory_space=pltpu.SMEM
            ),
            pl.BlockSpec((N, D), lambda i: (0, 0)),
            pl.BlockSpec((_UTSD, W1.shape[1]), lambda i: (0, 0)),
            pl.BlockSpec((1, W1.shape[1]), lambda i: (0, 0)),
            pl.BlockSpec((W2.shape[0], 1), lambda i: (0, 0)),
            pl.BlockSpec((1, 1), lambda i: (0, 0)),
        ],
        out_specs=pl.BlockSpec((1, _B, 1), lambda i: (i, 0, 0)),
        out_shape=jax.ShapeDtypeStruct((nblk, _B, 1), jnp.float32),
        scratch_shapes=[pltpu.VMEM((_B * _M, D), jnp.float32)],
        compiler_params=pltpu.CompilerParams(
            dimension_semantics=("arbitrary",)
        ),
    )(htab, x, W1, b1.reshape(1, -1), W2, b2.reshape(1, 1))
    return scores.reshape(-1)[:N]


# speedup vs baseline: 1.6840x; 1.6135x over previous
import jax
import jax.numpy as jnp
from jax.experimental import pallas as pl
from jax.experimental.pallas import tpu as pltpu

_UTSD = 8
_MAXD = 32
_K = 4
_M = _MAXD + 1
_BIG = 1e9
_B = 8
_BM = _B * _M


def _dotT(a, b):
    return jax.lax.dot_general(
        a, b, (((0,), (0,)), ((), ())), preferred_element_type=jnp.float32
    )


def _dot(a, b):
    return jax.lax.dot_general(
        a, b, (((1,), (0,)), ((), ())), preferred_element_type=jnp.float32
    )


def _uts_body(htab_ref, x_ref, eye_ref, selfbig_ref, sseg_ref, mrow_ref,
              mcol_ref, w1_ref, b1_ref, w2_ref, b2_ref, out_ref, h_ref):
    for b in range(_B):
        for j in range(_M):
            idx = htab_ref[0, b, j]
            h_ref[pl.ds(b * _M + j, 1), :] = x_ref[pl.ds(idx, 1), :]

    h = h_ref[...]
    gram = jax.lax.dot_general(
        h, h, (((1,), (1,)), ((), ())), preferred_element_type=jnp.float32
    )
    eye = eye_ref[...]
    ge = gram * eye
    ones_col = jnp.full((_BM, 1), 1.0, dtype=jnp.float32)
    ones_row = jnp.full((1, _BM), 1.0, dtype=jnp.float32)
    dcol = _dot(ge, ones_col)
    drow = _dot(ones_row, ge)
    mrow = mrow_ref[0]

    d2_blocks = []
    mrow_blocks = []
    for b in range(_B):
        lo = b * _M
        gb = jax.lax.slice(gram, (lo, lo), (lo + _M, lo + _M))
        dc = jax.lax.slice(dcol, (lo, 0), (lo + _M, 1))
        dr = jax.lax.slice(drow, (0, lo), (1, lo + _M))
        d2_blocks.append(dc + dr - 2.0 * gb)
        mr = jax.lax.slice(mrow, (0, lo), (1, lo + _M))
        mrow_blocks.append(jnp.broadcast_to(mr, (_M, _M)))
    d2 = jnp.concatenate(d2_blocks, axis=0)
    mrow_band = jnp.concatenate(mrow_blocks, axis=0)

    dist = jnp.sqrt(jnp.maximum(d2, 0.0) + 1e-12)
    mcol01 = mcol_ref[0]
    mcolbig = (1.0 - mcol01) * _BIG
    dm = dist + selfbig_ref[...] + mrow_band + mcolbig

    jj = jax.lax.broadcasted_iota(jnp.int32, (_BM, _M), 1)
    kds = []
    for k in range(_K):
        m = jnp.min(dm, axis=1, keepdims=True)
        kds.append(m)
        if k < _K - 1:
            sel = jnp.where(dm <= m, jj, _M)
            am = jnp.min(sel, axis=1, keepdims=True)
            dm = jnp.where(jj == am, _BIG, dm)
    kd = jnp.concatenate(kds, axis=1)

    wf = (kd < _BIG * 0.5).astype(jnp.float32) * mcol01
    kdw = wf * kd
    pk_raw = jnp.sum(wf, axis=1, keepdims=True)
    skdw = jnp.sum(kdw, axis=1, keepdims=True)
    skd2w = jnp.sum(kdw * kd, axis=1, keepdims=True)
    pm = skdw / jnp.maximum(pk_raw, 1.0)
    mpm = mcol01 * pm
    nd = jax.lax.slice(kd, (0, 0), (_BM, 1))
    ndokf = mcol01 * (nd < _BIG * 0.5).astype(jnp.float32)
    ndw = ndokf * nd
    q = jnp.concatenate(
        [pk_raw, skdw, skd2w, mpm, mpm * pm, ndw, ndw * nd, mcol01, ndokf],
        axis=1,
    )
    seg = _dot(sseg_ref[...], q)

    def col(i):
        return jax.lax.slice(seg, (0, i), (_B, i + 1))

    wsum = jnp.maximum(col(0), 1.0)
    s_mean = col(1) / wsum
    s_var = jnp.maximum(col(2) / wsum - s_mean * s_mean, 0.0)
    s_std = jnp.sqrt(s_var + 1e-12)
    cf = jnp.maximum(col(7), 1.0)
    p_mean = col(3) / cf
    p_var = jnp.maximum(col(4) / cf - p_mean * p_mean, 0.0)
    p_std = jnp.sqrt(p_var + 1e-12)
    n_mean = col(5) / cf
    n_var = jnp.maximum(
        (col(6) - n_mean * (2.0 * col(5) - n_mean * col(8))) / cf, 0.0
    )
    n_std = jnp.sqrt(n_var + 1e-12)

    rowmin = jnp.where(wf[:, 0:1] > 0.5, nd, _BIG)
    rowmax = jnp.max(jnp.where(wf > 0.5, kd, -_BIG), axis=1, keepdims=True)
    pairv = jnp.concatenate([rowmin, -rowmax], axis=1)
    tp = _dotT(pairv, eye)
    mins = []
    for b in range(_B):
        tb = jax.lax.slice(tp, (0, b * _M), (2, b * _M + _M))
        mins.append(jnp.min(tb, axis=1, keepdims=True))
    mm2 = jnp.concatenate(mins, axis=1)
    i2 = (jax.lax.broadcasted_iota(jnp.int32, (2, 2), 0)
          == jax.lax.broadcasted_iota(jnp.int32, (2, 2), 1)).astype(jnp.float32)
    smm = _dotT(mm2, i2)
    s_min = jax.lax.slice(smm, (0, 0), (_B, 1))
    s_max = -jax.lax.slice(smm, (0, 1), (_B, 2))

    okf = (cf >= 2.5).astype(jnp.float32)
    uts = jnp.concatenate(
        [s_mean, s_std, s_min, s_max, p_mean, p_std, n_mean, n_std], axis=1
    ) * okf

    h1 = jnp.maximum(_dot(uts, w1_ref[...]) + b1_ref[...], 0.0)
    out_ref[0] = _dot(h1, w2_ref[...]) + b2_ref[...]


def kernel(x, edge_index, W1, b1, W2, b2):
    N, D = x.shape
    E = edge_index.shape[1]
    ei = edge_index.astype(jnp.int32)
    s_all = jnp.concatenate([ei[0], ei[1]])
    d_all = jnp.concatenate([ei[1], ei[0]])
    ss, dd = jax.lax.sort((s_all, d_all), num_keys=1, is_stable=True)
    counts = jnp.bincount(s_all, length=N).astype(jnp.int32)
    starts = (jnp.cumsum(counts) - counts).astype(jnp.int32)
    rank = jnp.arange(2 * E, dtype=jnp.int32) - starts[ss]
    valid = rank < _MAXD
    row = jnp.where(valid, ss, N)
    col = jnp.where(valid, jnp.minimum(rank, _MAXD - 1), 0)
    nbr = (
        jnp.zeros((N + 1, _MAXD), dtype=jnp.int32)
        .at[row, col]
        .set(jnp.where(valid, dd, 0))[:N]
    )
    cnt = 1 + jnp.minimum(counts, _MAXD)
    hood = jnp.concatenate(
        [jnp.arange(N, dtype=jnp.int32)[:, None], nbr, cnt[:, None]], axis=1
    )

    pad = (-N) % _B
    if pad:
        filler = jnp.zeros((pad, _M + 1), dtype=jnp.int32).at[:, _M].set(1)
        hood = jnp.concatenate([hood, filler], axis=0)
    npad = hood.shape[0]
    nblk = npad // _B
    htab = hood.reshape(nblk, _B, _M + 1)

    maskf = (
        jnp.arange(_M, dtype=jnp.int32)[None, :] < hood[:, _M][:, None]
    ).astype(jnp.float32)
    mrow_all = ((1.0 - maskf) * _BIG).reshape(nblk, 1, _BM)
    mcol_all = maskf.reshape(nblk, _BM, 1)

    eye_c = jnp.eye(_BM, dtype=jnp.float32)
    r33 = jnp.arange(_BM, dtype=jnp.int32) % _M
    selfbig_c = (r33[:, None] == jnp.arange(_M, dtype=jnp.int32)[None, :]
                 ).astype(jnp.float32) * _BIG
    sseg_c = (
        jnp.arange(_B, dtype=jnp.int32)[:, None]
        == (jnp.arange(_BM, dtype=jnp.int32) // _M)[None, :]
    ).astype(jnp.float32)

    const_spec = lambda shp: pl.BlockSpec(shp, lambda i: tuple(0 for _ in shp))
    scores = pl.pallas_call(
        _uts_body,
        grid=(nblk,),
        in_specs=[
            pl.BlockSpec(
                (1, _B, _M + 1), lambda i: (i, 0, 0), memory_space=pltpu.SMEM
            ),
            const_spec((N, D)),
            const_spec((_BM, _BM)),
            const_spec((_BM, _M)),
            const_spec((_B, _BM)),
            pl.BlockSpec((1, 1, _BM), lambda i: (i, 0, 0)),
            pl.BlockSpec((1, _BM, 1), lambda i: (i, 0, 0)),
            const_spec((_UTSD, W1.shape[1])),
            const_spec((1, W1.shape[1])),
            const_spec((W2.shape[0], 1)),
            const_spec((1, 1)),
        ],
        out_specs=pl.BlockSpec((1, _B, 1), lambda i: (i, 0, 0)),
        out_shape=jax.ShapeDtypeStruct((nblk, _B, 1), jnp.float32),
        scratch_shapes=[pltpu.VMEM((_BM, D), jnp.float32)],
        compiler_params=pltpu.CompilerParams(
            dimension_semantics=("arbitrary",)
        ),
    )(htab, x, eye_c, selfbig_c, sseg_c, mrow_all, mcol_all,
      W1, b1.reshape(1, -1), W2, b2.reshape(1, 1))
    return scores.reshape(-1)[:N]
